# baseline (speedup 1.0000x reference)
import functools

import jax
import jax.numpy as jnp
from jax import lax
from jax.experimental import pallas as pl
from jax.experimental.pallas import tpu as pltpu
from jax.experimental.pallas import tpu_sc as plsc

N, C, H, W, E, D = 8, 192, 56, 56, 16, 64
HW = H * W
K = 4
LOCS = N * HW
BLK = 512
NBLK = (HW + BLK - 1) // BLK

DBLK = 512
HW_SC = 1024
HW_TC = HW - HW_SC
OFFBLK = HW_SC // DBLK
NBLK_TC = (HW_TC + DBLK - 1) // DBLK

NCORES = 2
NSUB = 16
NWORK = NCORES * NSUB
TILES_PER_IMG = NWORK // N
PER_TILE = HW_SC // TILES_PER_IMG
LC = 8
NCH = PER_TILE // LC
PAD_PT = PER_TILE + 16


def _gate_body(x_ref, w_ref, b_ref, vals_ref, idx_ref, wm_ref):
    xb = x_ref[0]
    logits = lax.dot_general(w_ref[...], xb, (((1,), (1,)), ((), ())),
                             preferred_element_type=jnp.float32)
    logits = logits + b_ref[...]
    m = jnp.max(logits, axis=0, keepdims=True)
    p = jnp.exp(logits - m)
    rw = p / jnp.sum(p, axis=0, keepdims=True)
    ids = lax.broadcasted_iota(jnp.int32, (E, BLK), 0)
    cur = rw
    vrows, irows = [], []
    for _ in range(K):
        mval = jnp.max(cur, axis=0, keepdims=True)
        sel = jnp.min(jnp.where(cur == mval, ids, E), axis=0, keepdims=True)
        vrows.append(mval)
        irows.append(sel)
        cur = jnp.where(ids == sel, -jnp.inf, cur)
    vals_ref[0] = jnp.concatenate(vrows, axis=0)
    idx_ref[0] = jnp.concatenate(irows, axis=0)
    wm_ref[0] = jnp.where(cur == -jnp.inf, rw, 0.0)


def _gate(xt, gate_w, gate_b2):
    return pl.pallas_call(
        _gate_body,
        grid=(N, NBLK),
        in_specs=[
            pl.BlockSpec((1, BLK, C), lambda n, c: (n, c, 0)),
            pl.BlockSpec((E, C), lambda n, c: (0, 0)),
            pl.BlockSpec((E, 1), lambda n, c: (0, 0)),
        ],
        out_specs=[
            pl.BlockSpec((1, K, BLK), lambda n, c: (n, 0, c)),
            pl.BlockSpec((1, K, BLK), lambda n, c: (n, 0, c)),
            pl.BlockSpec((1, E, BLK), lambda n, c: (n, 0, c)),
        ],
        out_shape=[
            jax.ShapeDtypeStruct((N, K, HW), jnp.float32),
            jax.ShapeDtypeStruct((N, K, HW), jnp.int32),
            jax.ShapeDtypeStruct((N, E, HW), jnp.float32),
        ],
    )(xt, gate_w, gate_b2)


def _dense_body(w_ref, e_ref, o_ref):
    wv = w_ref[0]
    acc = None
    for e in range(E):
        t = e_ref[0, e] * wv[e][:, None]
        acc = t if acc is None else acc + t
    o_ref[0] = acc


def _dense(wmask, ex4):
    return pl.pallas_call(
        _dense_body,
        grid=(N, NBLK_TC),
        in_specs=[
            pl.BlockSpec((1, E, DBLK), lambda n, c: (n, 0, c + OFFBLK)),
            pl.BlockSpec((1, E, DBLK, D), lambda n, c: (n, 0, c + OFFBLK, 0)),
        ],
        out_specs=pl.BlockSpec((1, DBLK, D), lambda n, c: (n, c, 0)),
        out_shape=jax.ShapeDtypeStruct((N, HW_TC, D), jnp.float32),
    )(wmask, ex4)


def _route_body(idx_hbm, vals_hbm, experts_hbm, out_hbm,
                iv0, iv1, iv2, iv3, vv0, vv1, vv2, vv3,
                ebuf, obuf, gsem0, gsem1, osem0, osem1):
    wid = lax.axis_index("s") * NCORES + lax.axis_index("c")
    loc0 = wid * PER_TILE
    n = wid // TILES_PER_IMG
    hw0 = (wid % TILES_PER_IMG) * PER_TILE
    gsems = (gsem0, gsem1)
    osems = (osem0, osem1)
    idx_v = (iv0, iv1, iv2, iv3)
    vals_v = (vv0, vv1, vv2, vv3)

    for j in range(K):
        off = (n * K + j) * HW + hw0
        pltpu.sync_copy(idx_hbm.at[pl.ds(off, PER_TILE)],
                        idx_v[j].at[pl.ds(0, PER_TILE)])
        pltpu.sync_copy(vals_hbm.at[pl.ds(off, PER_TILE)],
                        vals_v[j].at[pl.ds(0, PER_TILE)])

    def slab_src(ci):
        return experts_hbm.at[n, :, pl.ds(hw0 + ci * LC, LC), :]

    def fire(ci, b):
        pltpu.async_copy(slab_src(ci), ebuf.at[b], gsems[b])

    fire(0, 0)
    fire(1, 1)

    def pair(g, carry):
        for b in range(2):
            ci = 2 * g + b
            pltpu.make_async_copy(slab_src(ci), ebuf.at[b], gsems[b]).wait()

            @pl.when(ci >= 2)
            def _():
                pltpu.make_async_copy(
                    obuf.at[b],
                    out_hbm.at[pl.ds(loc0 + (ci - 2) * LC, LC)],
                    osems[b],
                ).wait()

            base = ci * LC
            wv = [vals_v[j][pl.ds(base, 16)] for j in range(K)]
            iv = [idx_v[j][pl.ds(base, 16)] for j in range(K)]
            for r in range(LC):
                for dd in range(D // 16):
                    acc = None
                    for j in range(K):
                        row = ebuf[b, iv[j][r], r, pl.ds(dd * 16, 16)]
                        t = row * wv[j][r]
                        acc = t if acc is None else acc + t
                    obuf[b, r, pl.ds(dd * 16, 16)] = acc
            pltpu.async_copy(
                obuf.at[b],
                out_hbm.at[pl.ds(loc0 + ci * LC, LC)],
                osems[b],
            )

            @pl.when(ci + 2 < NCH)
            def _():
                pltpu.async_copy(slab_src(ci + 2), ebuf.at[b], gsems[b])

        return carry

    lax.fori_loop(0, NCH // 2, pair, 0)
    for b in range(2):
        ci = NCH - 2 + b
        pltpu.make_async_copy(
            obuf.at[b],
            out_hbm.at[pl.ds(loc0 + ci * LC, LC)],
            osems[b],
        ).wait()


@functools.lru_cache(maxsize=1)
def _build_route():
    return pl.kernel(
        _route_body,
        out_type=jax.ShapeDtypeStruct((N * HW_SC, D), jnp.float32),
        mesh=plsc.VectorSubcoreMesh(core_axis_name="c", subcore_axis_name="s"),
        compiler_params=pltpu.CompilerParams(use_tc_tiling_on_sc=True),
        scratch_types=(
            [pltpu.VMEM((PAD_PT,), jnp.int32) for _ in range(K)]
            + [pltpu.VMEM((PAD_PT,), jnp.float32) for _ in range(K)]
            + [
                pltpu.VMEM((2, E, LC, D), jnp.float32),
                pltpu.VMEM((2, LC, D), jnp.float32),
                pltpu.SemaphoreType.DMA,
                pltpu.SemaphoreType.DMA,
                pltpu.SemaphoreType.DMA,
                pltpu.SemaphoreType.DMA,
            ]
        ),
    )


def kernel(x, experts, gate_w, gate_b):
    xt = x.transpose(0, 2, 3, 1).reshape(N, HW, C)
    vals, sel, wmask = _gate(xt, gate_w, gate_b.reshape(E, 1))
    ex4 = experts.reshape(N, E, HW, D)
    route = _build_route()
    sel1, vals1 = sel.reshape(-1), vals.reshape(-1)
    out_sc = route(sel1, vals1, ex4)
    out_tc = _dense(wmask, ex4)
    del route
    out = jnp.concatenate([out_sc.reshape(N, HW_SC, D), out_tc], axis=1)
    return out.reshape(N, H, W, D)


# speedup vs baseline: 1.0102x; 1.0102x over previous
import functools

import jax
import jax.numpy as jnp
from jax import lax
from jax.experimental import pallas as pl
from jax.experimental.pallas import tpu as pltpu
from jax.experimental.pallas import tpu_sc as plsc

N, C, H, W, E, D = 8, 192, 56, 56, 16, 64
HW = H * W
K = 4
LOCS = N * HW
BLK = 512
NBLK = (HW + BLK - 1) // BLK

DBLK = 512
HW_SC = 2048
HW_TC = HW - HW_SC
OFFBLK = HW_SC // DBLK
NBLK_TC = (HW_TC + DBLK - 1) // DBLK

NCORES = 2
NSUB = 16
NWORK = NCORES * NSUB
TILES_PER_IMG = NWORK // N
PER_TILE = HW_SC // TILES_PER_IMG
LC = 8
NCH = PER_TILE // LC
PAD_PT = PER_TILE + 16


def _gate_body(x_ref, w_ref, b_ref, vals_ref, idx_ref, wm_ref):
    xb = x_ref[0]
    logits = lax.dot_general(w_ref[...], xb, (((1,), (1,)), ((), ())),
                             preferred_element_type=jnp.float32)
    logits = logits + b_ref[...]
    m = jnp.max(logits, axis=0, keepdims=True)
    p = jnp.exp(logits - m)
    rw = p / jnp.sum(p, axis=0, keepdims=True)
    ids = lax.broadcasted_iota(jnp.int32, (E, BLK), 0)
    cur = rw
    vrows, irows = [], []
    for _ in range(K):
        mval = jnp.max(cur, axis=0, keepdims=True)
        sel = jnp.min(jnp.where(cur == mval, ids, E), axis=0, keepdims=True)
        vrows.append(mval)
        irows.append(sel)
        cur = jnp.where(ids == sel, -jnp.inf, cur)
    vals_ref[0] = jnp.concatenate(vrows, axis=0)
    idx_ref[0] = jnp.concatenate(irows, axis=0)
    wm_ref[0] = jnp.where(cur == -jnp.inf, rw, 0.0)


def _gate(xt, gate_w, gate_b2):
    return pl.pallas_call(
        _gate_body,
        grid=(N, NBLK),
        in_specs=[
            pl.BlockSpec((1, BLK, C), lambda n, c: (n, c, 0)),
            pl.BlockSpec((E, C), lambda n, c: (0, 0)),
            pl.BlockSpec((E, 1), lambda n, c: (0, 0)),
        ],
        out_specs=[
            pl.BlockSpec((1, K, BLK), lambda n, c: (n, 0, c)),
            pl.BlockSpec((1, K, BLK), lambda n, c: (n, 0, c)),
            pl.BlockSpec((1, E, BLK), lambda n, c: (n, 0, c)),
        ],
        out_shape=[
            jax.ShapeDtypeStruct((N, K, HW), jnp.float32),
            jax.ShapeDtypeStruct((N, K, HW), jnp.int32),
            jax.ShapeDtypeStruct((N, E, HW), jnp.float32),
        ],
    )(xt, gate_w, gate_b2)


def _dense_body(w_ref, e_ref, o_ref):
    wv = w_ref[0]
    acc = None
    for e in range(E):
        t = e_ref[0, e] * wv[e][:, None]
        acc = t if acc is None else acc + t
    o_ref[0] = acc


def _dense(wmask, ex4):
    return pl.pallas_call(
        _dense_body,
        grid=(N, NBLK_TC),
        in_specs=[
            pl.BlockSpec((1, E, DBLK), lambda n, c: (n, 0, c + OFFBLK)),
            pl.BlockSpec((1, E, DBLK, D), lambda n, c: (n, 0, c + OFFBLK, 0)),
        ],
        out_specs=pl.BlockSpec((1, DBLK, D), lambda n, c: (n, c, 0)),
        out_shape=jax.ShapeDtypeStruct((N, HW_TC, D), jnp.float32),
    )(wmask, ex4)


def _route_body(idx_hbm, vals_hbm, experts_hbm, out_hbm,
                iv0, iv1, iv2, iv3, vv0, vv1, vv2, vv3,
                ebuf, obuf, gsem0, gsem1, osem0, osem1):
    wid = lax.axis_index("s") * NCORES + lax.axis_index("c")
    loc0 = wid * PER_TILE
    n = wid // TILES_PER_IMG
    hw0 = (wid % TILES_PER_IMG) * PER_TILE
    gsems = (gsem0, gsem1)
    osems = (osem0, osem1)
    idx_v = (iv0, iv1, iv2, iv3)
    vals_v = (vv0, vv1, vv2, vv3)

    for j in range(K):
        off = (n * K + j) * HW + hw0
        pltpu.sync_copy(idx_hbm.at[pl.ds(off, PER_TILE)],
                        idx_v[j].at[pl.ds(0, PER_TILE)])
        pltpu.sync_copy(vals_hbm.at[pl.ds(off, PER_TILE)],
                        vals_v[j].at[pl.ds(0, PER_TILE)])

    def slab_src(ci):
        return experts_hbm.at[n, :, pl.ds(hw0 + ci * LC, LC), :]

    def fire(ci, b):
        pltpu.async_copy(slab_src(ci), ebuf.at[b], gsems[b])

    fire(0, 0)
    fire(1, 1)

    def pair(g, carry):
        for b in range(2):
            ci = 2 * g + b
            pltpu.make_async_copy(slab_src(ci), ebuf.at[b], gsems[b]).wait()

            @pl.when(ci >= 2)
            def _():
                pltpu.make_async_copy(
                    obuf.at[b],
                    out_hbm.at[pl.ds(loc0 + (ci - 2) * LC, LC)],
                    osems[b],
                ).wait()

            base = ci * LC
            wv = [vals_v[j][pl.ds(base, 16)] for j in range(K)]
            iv = [idx_v[j][pl.ds(base, 16)] for j in range(K)]
            for r in range(LC):
                for dd in range(D // 16):
                    acc = None
                    for j in range(K):
                        row = ebuf[b, iv[j][r], r, pl.ds(dd * 16, 16)]
                        t = row * wv[j][r]
                        acc = t if acc is None else acc + t
                    obuf[b, r, pl.ds(dd * 16, 16)] = acc
            pltpu.async_copy(
                obuf.at[b],
                out_hbm.at[pl.ds(loc0 + ci * LC, LC)],
                osems[b],
            )

            @pl.when(ci + 2 < NCH)
            def _():
                pltpu.async_copy(slab_src(ci + 2), ebuf.at[b], gsems[b])

        return carry

    lax.fori_loop(0, NCH // 2, pair, 0)
    for b in range(2):
        ci = NCH - 2 + b
        pltpu.make_async_copy(
            obuf.at[b],
            out_hbm.at[pl.ds(loc0 + ci * LC, LC)],
            osems[b],
        ).wait()


@functools.lru_cache(maxsize=1)
def _build_route():
    return pl.kernel(
        _route_body,
        out_type=jax.ShapeDtypeStruct((N * HW_SC, D), jnp.float32),
        mesh=plsc.VectorSubcoreMesh(core_axis_name="c", subcore_axis_name="s"),
        compiler_params=pltpu.CompilerParams(use_tc_tiling_on_sc=True),
        scratch_types=(
            [pltpu.VMEM((PAD_PT,), jnp.int32) for _ in range(K)]
            + [pltpu.VMEM((PAD_PT,), jnp.float32) for _ in range(K)]
            + [
                pltpu.VMEM((2, E, LC, D), jnp.float32),
                pltpu.VMEM((2, LC, D), jnp.float32),
                pltpu.SemaphoreType.DMA,
                pltpu.SemaphoreType.DMA,
                pltpu.SemaphoreType.DMA,
                pltpu.SemaphoreType.DMA,
            ]
        ),
    )


def kernel(x, experts, gate_w, gate_b):
    xt = x.transpose(0, 2, 3, 1).reshape(N, HW, C)
    vals, sel, wmask = _gate(xt, gate_w, gate_b.reshape(E, 1))
    ex4 = experts.reshape(N, E, HW, D)
    route = _build_route()
    sel1, vals1 = sel.reshape(-1), vals.reshape(-1)
    out_sc = route(sel1, vals1, ex4)
    out_tc = _dense(wmask, ex4)
    del route
    out = jnp.concatenate([out_sc.reshape(N, HW_SC, D), out_tc], axis=1)
    return out.reshape(N, H, W, D)


# speedup vs baseline: 1.0519x; 1.0413x over previous
import functools

import jax
import jax.numpy as jnp
from jax import lax
from jax.experimental import pallas as pl
from jax.experimental.pallas import tpu as pltpu
from jax.experimental.pallas import tpu_sc as plsc

N, C, H, W, E, D = 8, 192, 56, 56, 16, 64
HW = H * W
K = 4
LOCS = N * HW
BLK = 512
NBLK = (HW + BLK - 1) // BLK

DBLK = 512
HW_SC = 1536
HW_TC = HW - HW_SC
OFFBLK = HW_SC // DBLK
NBLK_TC = (HW_TC + DBLK - 1) // DBLK

NCORES = 2
NSUB = 16
NWORK = NCORES * NSUB
TILES_PER_IMG = NWORK // N
PER_TILE = HW_SC // TILES_PER_IMG
LC = 8
NCH = PER_TILE // LC
WPAD = 512


def _gate_body(x_ref, w_ref, b_ref, wm_ref):
    xb = x_ref[0]
    logits = lax.dot_general(w_ref[...], xb, (((1,), (1,)), ((), ())),
                             preferred_element_type=jnp.float32)
    logits = logits + b_ref[...]
    m = jnp.max(logits, axis=0, keepdims=True)
    p = jnp.exp(logits - m)
    rw = p / jnp.sum(p, axis=0, keepdims=True)
    ids = lax.broadcasted_iota(jnp.int32, (E, BLK), 0)
    cur = rw
    for _ in range(K):
        mval = jnp.max(cur, axis=0, keepdims=True)
        sel = jnp.min(jnp.where(cur == mval, ids, E), axis=0, keepdims=True)
        cur = jnp.where(ids == sel, -jnp.inf, cur)
    wm_ref[0] = jnp.where(cur == -jnp.inf, rw, 0.0)


def _gate(xt, gate_w, gate_b2):
    return pl.pallas_call(
        _gate_body,
        grid=(N, NBLK),
        in_specs=[
            pl.BlockSpec((1, BLK, C), lambda n, c: (n, c, 0)),
            pl.BlockSpec((E, C), lambda n, c: (0, 0)),
            pl.BlockSpec((E, 1), lambda n, c: (0, 0)),
        ],
        out_specs=pl.BlockSpec((1, E, BLK), lambda n, c: (n, 0, c)),
        out_shape=jax.ShapeDtypeStruct((N, E, HW), jnp.float32),
    )(xt, gate_w, gate_b2)


def _dense_body(w_ref, e_ref, o_ref):
    wv = w_ref[0]
    acc = None
    for e in range(E):
        t = e_ref[0, e] * wv[e][:, None]
        acc = t if acc is None else acc + t
    o_ref[0] = acc


def _dense(wmask, ex4):
    return pl.pallas_call(
        _dense_body,
        grid=(N, NBLK_TC),
        in_specs=[
            pl.BlockSpec((1, E, DBLK), lambda n, c: (n, 0, c + OFFBLK)),
            pl.BlockSpec((1, E, DBLK, D), lambda n, c: (n, 0, c + OFFBLK, 0)),
        ],
        out_specs=pl.BlockSpec((1, DBLK, D), lambda n, c: (n, c + OFFBLK, 0)),
        out_shape=jax.ShapeDtypeStruct((N, HW, D), jnp.float32),
    )(wmask, ex4)


def _route_body(wm_hbm, experts_hbm, out_hbm,
                wbuf, ebuf, obuf, gsem0, gsem1, osem0, osem1):
    wid = lax.axis_index("s") * NCORES + lax.axis_index("c")
    loc0 = wid * PER_TILE
    n = wid // TILES_PER_IMG
    hw0 = (wid % TILES_PER_IMG) * PER_TILE
    gsems = (gsem0, gsem1)
    osems = (osem0, osem1)

    pltpu.sync_copy(wm_hbm.at[n, :, pl.ds(hw0, PER_TILE)],
                    wbuf.at[:, pl.ds(0, PER_TILE)])

    def slab_src(ci):
        return experts_hbm.at[n, :, pl.ds(hw0 + ci * LC, LC), :]

    def fire(ci, b):
        pltpu.async_copy(slab_src(ci), ebuf.at[b], gsems[b])

    fire(0, 0)
    fire(1, 1)

    def pair(g, carry):
        wv = [wbuf[e, pl.ds(g * 16, 16)] for e in range(E)]
        for b in range(2):
            ci = 2 * g + b
            pltpu.make_async_copy(slab_src(ci), ebuf.at[b], gsems[b]).wait()

            @pl.when(ci >= 2)
            def _():
                pltpu.make_async_copy(
                    obuf.at[b],
                    out_hbm.at[pl.ds(loc0 + (ci - 2) * LC, LC)],
                    osems[b],
                ).wait()

            for r in range(LC):
                for dd in range(D // 16):
                    acc = None
                    for e in range(E):
                        t = ebuf[b, e, r, pl.ds(dd * 16, 16)] * wv[e][b * LC + r]
                        acc = t if acc is None else acc + t
                    obuf[b, r, pl.ds(dd * 16, 16)] = acc
            pltpu.async_copy(
                obuf.at[b],
                out_hbm.at[pl.ds(loc0 + ci * LC, LC)],
                osems[b],
            )

            @pl.when(ci + 2 < NCH)
            def _():
                pltpu.async_copy(slab_src(ci + 2), ebuf.at[b], gsems[b])

        return carry

    lax.fori_loop(0, NCH // 2, pair, 0)
    for b in range(2):
        ci = NCH - 2 + b
        pltpu.make_async_copy(
            obuf.at[b],
            out_hbm.at[pl.ds(loc0 + ci * LC, LC)],
            osems[b],
        ).wait()


@functools.lru_cache(maxsize=1)
def _build_route():
    return pl.kernel(
        _route_body,
        out_type=jax.ShapeDtypeStruct((N * HW_SC, D), jnp.float32),
        mesh=plsc.VectorSubcoreMesh(core_axis_name="c", subcore_axis_name="s"),
        compiler_params=pltpu.CompilerParams(use_tc_tiling_on_sc=True),
        scratch_types=[
            pltpu.VMEM((E, WPAD), jnp.float32),
            pltpu.VMEM((2, E, LC, D), jnp.float32),
            pltpu.VMEM((2, LC, D), jnp.float32),
            pltpu.SemaphoreType.DMA,
            pltpu.SemaphoreType.DMA,
            pltpu.SemaphoreType.DMA,
            pltpu.SemaphoreType.DMA,
        ],
    )


def kernel(x, experts, gate_w, gate_b):
    xt = x.transpose(0, 2, 3, 1).reshape(N, HW, C)
    wmask = _gate(xt, gate_w, gate_b.reshape(E, 1))
    ex4 = experts.reshape(N, E, HW, D)
    out_sc = _build_route()(wmask, ex4)
    out_tc = _dense(wmask, ex4)
    out = lax.dynamic_update_slice(
        out_tc, out_sc.reshape(N, HW_SC, D), (0, 0, 0))
    return out.reshape(N, H, W, D)


# speedup vs baseline: 1.0819x; 1.0285x over previous
import functools

import jax
import jax.numpy as jnp
from jax import lax
from jax.experimental import pallas as pl
from jax.experimental.pallas import tpu as pltpu
from jax.experimental.pallas import tpu_sc as plsc

N, C, H, W, E, D = 8, 192, 56, 56, 16, 64
HW = H * W
K = 4
LOCS = N * HW
BLK = 512
NBLK = (HW + BLK - 1) // BLK

DBLK = 512
HW_SC = 1024
HW_TC = HW - HW_SC
OFFBLK = HW_SC // DBLK
NBLK_TC = (HW_TC + DBLK - 1) // DBLK

NCORES = 2
NSUB = 16
NWORK = NCORES * NSUB
TILES_PER_IMG = NWORK // N
PER_TILE = HW_SC // TILES_PER_IMG
LC = 8
NCH = PER_TILE // LC
WPAD = 512


def _gate_body(x_ref, w_ref, b_ref, wm_ref):
    xb = x_ref[0]
    logits = lax.dot_general(w_ref[...], xb, (((1,), (1,)), ((), ())),
                             preferred_element_type=jnp.float32)
    logits = logits + b_ref[...]
    m = jnp.max(logits, axis=0, keepdims=True)
    p = jnp.exp(logits - m)
    rw = p / jnp.sum(p, axis=0, keepdims=True)
    ids = lax.broadcasted_iota(jnp.int32, (E, BLK), 0)
    cur = rw
    for _ in range(K):
        mval = jnp.max(cur, axis=0, keepdims=True)
        sel = jnp.min(jnp.where(cur == mval, ids, E), axis=0, keepdims=True)
        cur = jnp.where(ids == sel, -jnp.inf, cur)
    wm_ref[0] = jnp.where(cur == -jnp.inf, rw, 0.0)


def _gate(xt, gate_w, gate_b2):
    return pl.pallas_call(
        _gate_body,
        grid=(N, NBLK),
        in_specs=[
            pl.BlockSpec((1, BLK, C), lambda n, c: (n, c, 0)),
            pl.BlockSpec((E, C), lambda n, c: (0, 0)),
            pl.BlockSpec((E, 1), lambda n, c: (0, 0)),
        ],
        out_specs=pl.BlockSpec((1, E, BLK), lambda n, c: (n, 0, c)),
        out_shape=jax.ShapeDtypeStruct((N, E, HW), jnp.float32),
    )(xt, gate_w, gate_b2)


def _dense_body(w_ref, e_ref, o_ref):
    wv = w_ref[0]
    acc = None
    for e in range(E):
        t = e_ref[0, e] * wv[e][:, None]
        acc = t if acc is None else acc + t
    o_ref[0] = acc


def _dense(wmask, ex4):
    return pl.pallas_call(
        _dense_body,
        grid=(N, NBLK_TC),
        in_specs=[
            pl.BlockSpec((1, E, DBLK), lambda n, c: (n, 0, c + OFFBLK)),
            pl.BlockSpec((1, E, DBLK, D), lambda n, c: (n, 0, c + OFFBLK, 0)),
        ],
        out_specs=pl.BlockSpec((1, DBLK, D), lambda n, c: (n, c + OFFBLK, 0)),
        out_shape=jax.ShapeDtypeStruct((N, HW, D), jnp.float32),
    )(wmask, ex4)


def _route_body(wm_hbm, experts_hbm, out_hbm,
                wbuf, ebuf, obuf, gsem0, gsem1, osem0, osem1):
    wid = lax.axis_index("s") * NCORES + lax.axis_index("c")
    loc0 = wid * PER_TILE
    n = wid // TILES_PER_IMG
    hw0 = (wid % TILES_PER_IMG) * PER_TILE
    gsems = (gsem0, gsem1)
    osems = (osem0, osem1)

    pltpu.sync_copy(wm_hbm.at[n, :, pl.ds(hw0, PER_TILE)],
                    wbuf.at[:, pl.ds(0, PER_TILE)])

    def slab_src(ci):
        return experts_hbm.at[n, :, pl.ds(hw0 + ci * LC, LC), :]

    def fire(ci, b):
        pltpu.async_copy(slab_src(ci), ebuf.at[b], gsems[b])

    fire(0, 0)
    fire(1, 1)

    def pair(g, carry):
        wv = [wbuf[e, pl.ds(g * 16, 16)] for e in range(E)]
        for b in range(2):
            ci = 2 * g + b
            pltpu.make_async_copy(slab_src(ci), ebuf.at[b], gsems[b]).wait()

            @pl.when(ci >= 2)
            def _():
                pltpu.make_async_copy(
                    obuf.at[b],
                    out_hbm.at[pl.ds(loc0 + (ci - 2) * LC, LC)],
                    osems[b],
                ).wait()

            for r in range(LC):
                for dd in range(D // 16):
                    acc = None
                    for e in range(E):
                        t = ebuf[b, e, r, pl.ds(dd * 16, 16)] * wv[e][b * LC + r]
                        acc = t if acc is None else acc + t
                    obuf[b, r, pl.ds(dd * 16, 16)] = acc
            pltpu.async_copy(
                obuf.at[b],
                out_hbm.at[pl.ds(loc0 + ci * LC, LC)],
                osems[b],
            )

            @pl.when(ci + 2 < NCH)
            def _():
                pltpu.async_copy(slab_src(ci + 2), ebuf.at[b], gsems[b])

        return carry

    lax.fori_loop(0, NCH // 2, pair, 0)
    for b in range(2):
        ci = NCH - 2 + b
        pltpu.make_async_copy(
            obuf.at[b],
            out_hbm.at[pl.ds(loc0 + ci * LC, LC)],
            osems[b],
        ).wait()


@functools.lru_cache(maxsize=1)
def _build_route():
    return pl.kernel(
        _route_body,
        out_type=jax.ShapeDtypeStruct((N * HW_SC, D), jnp.float32),
        mesh=plsc.VectorSubcoreMesh(core_axis_name="c", subcore_axis_name="s"),
        compiler_params=pltpu.CompilerParams(use_tc_tiling_on_sc=True),
        scratch_types=[
            pltpu.VMEM((E, WPAD), jnp.float32),
            pltpu.VMEM((2, E, LC, D), jnp.float32),
            pltpu.VMEM((2, LC, D), jnp.float32),
            pltpu.SemaphoreType.DMA,
            pltpu.SemaphoreType.DMA,
            pltpu.SemaphoreType.DMA,
            pltpu.SemaphoreType.DMA,
        ],
    )


def kernel(x, experts, gate_w, gate_b):
    xt = x.transpose(0, 2, 3, 1).reshape(N, HW, C)
    wmask = _gate(xt, gate_w, gate_b.reshape(E, 1))
    ex4 = experts.reshape(N, E, HW, D)
    out_sc = _build_route()(wmask, ex4)
    out_tc = _dense(wmask, ex4)
    out = lax.dynamic_update_slice(
        out_tc, out_sc.reshape(N, HW_SC, D), (0, 0, 0))
    return out.reshape(N, H, W, D)
